# baseline (speedup 1.0000x reference)
import functools

import jax
import jax.numpy as jnp
from jax import lax
from jax.experimental import pallas as pl
from jax.experimental.pallas import tpu as pltpu
from jax.experimental.pallas import tpu_sc as plsc

_NC = 2
_NS = 16
_L = 16
_NW = _NC * _NS

_RB = 8
_HB = _RB // 2


@functools.lru_cache(maxsize=None)
def _make_permute(rows: int, rows_sc: int, dim: int):
    assert rows_sc % _NW == 0 and dim % _L == 0
    rows_per_worker = rows_sc // _NW
    assert rows_per_worker % _RB == 0
    n_batches = rows_per_worker // _RB
    n_cols = dim // _L

    mesh = plsc.VectorSubcoreMesh(core_axis_name="c", subcore_axis_name="s")

    @functools.partial(
        pl.kernel,
        out_type=jax.ShapeDtypeStruct((rows_sc, dim), jnp.float32),
        mesh=mesh,
        compiler_params=pltpu.CompilerParams(needs_layout_passes=False),
        scratch_types=[
            pltpu.VMEM((dim,), jnp.int32),
            pltpu.VMEM((_RB, dim), jnp.float32),
            pltpu.VMEM((_RB, dim), jnp.float32),
            pltpu.VMEM((_HB, dim), jnp.float32),
            pltpu.VMEM((_HB, dim), jnp.float32),
            pltpu.SemaphoreType.DMA,
            pltpu.SemaphoreType.DMA,
            pltpu.SemaphoreType.DMA,
            pltpu.SemaphoreType.DMA,
        ],
    )
    def permute(x_hbm, perm_hbm, out_hbm, perm_v, in_v0, in_v1, out_v0,
                out_v1, in_s0, in_s1, out_s0, out_s1):
        wid = lax.axis_index("s") * _NC + lax.axis_index("c")
        row0 = wid * rows_per_worker
        pltpu.sync_copy(perm_hbm, perm_v)

        in_bufs, out_bufs = (in_v0, in_v1), (out_v0, out_v1)
        in_sems, out_sems = (in_s0, in_s1), (out_s0, out_s1)

        def in_copy(b, p):
            start = row0 + b * _RB
            return pltpu.make_async_copy(
                x_hbm.at[pl.ds(start, _RB)], in_bufs[p], in_sems[p])

        def out_copy(b, h):
            start = row0 + b * _RB + h * _HB
            return pltpu.make_async_copy(
                out_bufs[h], out_hbm.at[pl.ds(start, _HB)], out_sems[h])

        in_copy(0, 0).start()
        in_copy(1, 1).start()

        n_super = n_batches // 2

        def super_body(g, carry):
            for p in range(2):
                b = g * 2 + p
                in_copy(b, p).wait()
                in_v = in_bufs[p]

                for h in range(2):
                    @pl.when(b >= 1)
                    def _wait_out():
                        out_copy(b - 1, h).wait()

                    out_v = out_bufs[h]

                    @plsc.parallel_loop(0, n_cols, 1, unroll=8)
                    def col_body(k):
                        idx = perm_v[pl.ds(k * _L, _L)]
                        for r in range(_HB):
                            row_idx = jnp.full((_L,), h * _HB + r, jnp.int32)
                            vals = plsc.load_gather(in_v, [row_idx, idx])
                            out_v[r, pl.ds(k * _L, _L)] = vals

                    out_copy(b, h).start()

                @pl.when(g + 1 < n_super)
                def _next_in():
                    in_copy(b + 2, p).start()

            return carry

        lax.fori_loop(0, n_super, super_body, 0)
        out_copy(n_batches - 1, 0).wait()
        out_copy(n_batches - 1, 1).wait()

    return permute


@functools.lru_cache(maxsize=None)
def _make_tc_permute(rows: int, rows_tc: int, dim: int,
                     bm: int = 256, bn: int = 1024):
    assert rows_tc % bm == 0 and dim % bn == 0
    blk0 = (rows - rows_tc) // bm
    assert blk0 * bm == rows - rows_tc

    def body(x_ref, p_ref, o_ref):
        x_blk = x_ref[...]
        hi = x_blk.astype(jnp.bfloat16)
        lo = (x_blk - hi.astype(jnp.float32)).astype(jnp.bfloat16)
        p_blk = p_ref[...]
        dn = (((1,), (0,)), ((), ()))
        acc = lax.dot_general(hi, p_blk, dn, preferred_element_type=jnp.float32)
        acc += lax.dot_general(lo, p_blk, dn, preferred_element_type=jnp.float32)
        o_ref[...] = acc

    return pl.pallas_call(
        body,
        grid=(dim // bn, rows_tc // bm),
        in_specs=[
            pl.BlockSpec((bm, dim), lambda j, i: (blk0 + i, 0)),
            pl.BlockSpec((dim, bn), lambda j, i: (0, j)),
        ],
        out_specs=pl.BlockSpec((bm, bn), lambda j, i: (i, j)),
        out_shape=jax.ShapeDtypeStruct((rows_tc, dim), jnp.float32),
        compiler_params=pltpu.CompilerParams(
            vmem_limit_bytes=60 * 1024 * 1024),
    )


_TC_ROWS = 2048


def kernel(x, perm):
    rows, dim = x.shape
    perm = perm.astype(jnp.int32)
    rows_sc = rows - _TC_ROWS
    sc_out = _make_permute(rows, rows_sc, dim)(x, perm)
    pmat = (lax.broadcasted_iota(jnp.int32, (dim, dim), 0)
            == perm[None, :]).astype(jnp.bfloat16)
    tc_out = _make_tc_permute(rows, _TC_ROWS, dim)(x, pmat)
    out = jnp.concatenate([sc_out, tc_out], axis=0)
    return (out, 0)
